# baseline (speedup 1.0000x reference)
import functools

import jax
import jax.numpy as jnp
from jax import lax
from jax.experimental import pallas as pl
from jax.experimental.pallas import tpu as pltpu
from jax.experimental.pallas import tpu_sc as plsc

B, M, D = 8, 4096, 128
P, K2 = 2048, 10
K = K2 - 2
NC, NS, L = 2, 16, 16
BPC = B // NC
PPT = P // NS
CH = L
NCH = PPT // CH
EPS = 1e-6
DEPS2 = D * EPS * EPS
MARGIN = 0.5


def _consts_tc(src_ref, tgt_ref, ss_ref, su_ref, ts_ref, tv_ref):
    for tab, (ref, sref, uref) in enumerate(
            ((src_ref, ss_ref, su_ref), (tgt_ref, ts_ref, tv_ref))):
        x = ref[0]
        ones = jnp.ones((D,), jnp.float32)
        q = jax.lax.dot_general(
            x * x, ones, (((1,), (0,)), ((), ())),
            preferred_element_type=jnp.float32)
        sv = jax.lax.dot_general(
            x, ones, (((1,), (0,)), ((), ())),
            preferred_element_type=jnp.float32)
        scale = jnp.minimum(lax.rsqrt(jnp.maximum(q, 1e-30)), 1e12)
        es = (2.0 * EPS) * scale * sv
        sq = scale * scale * q
        sref[0, 0, :] = scale
        uref[0, 0, :] = sq + es if tab == 0 else sq - es


_row_consts = pl.pallas_call(
    _consts_tc,
    grid=(B,),
    in_specs=[pl.BlockSpec((1, M, D), lambda b: (b, 0, 0))] * 2,
    out_specs=[pl.BlockSpec((1, 1, M), lambda b: (b, 0, 0))] * 4,
    out_shape=[jax.ShapeDtypeStruct((B, 1, M), jnp.float32)] * 4,
)


def _rsqrt16(q):
    i = lax.bitcast_convert_type(q, jnp.int32)
    i = jnp.int32(0x5F3759DF) - (i >> 1)
    y = lax.bitcast_convert_type(i, jnp.float32)
    for _ in range(3):
        y = y * (1.5 - 0.5 * q * y * y)
    return y


_mesh = plsc.VectorSubcoreMesh(core_axis_name="c", subcore_axis_name="s")


@functools.partial(
    pl.kernel,
    out_type=jax.ShapeDtypeStruct((NC, NS, 2, L), jnp.float32),
    mesh=_mesh,
    compiler_params=pltpu.CompilerParams(needs_layout_passes=False),
    scratch_types=[
        pltpu.VMEM((BPC, M), jnp.float32),
        pltpu.VMEM((BPC, M), jnp.float32),
        pltpu.VMEM((BPC, M), jnp.float32),
        pltpu.VMEM((BPC, M), jnp.float32),
        pltpu.VMEM((K2, PPT), jnp.int32),
        pltpu.VMEM((2, K2, CH, D), jnp.float32),
        pltpu.VMEM((2, L), jnp.float32),
        pltpu.SemaphoreType.DMA,
        pltpu.SemaphoreType.DMA,
    ],
)
def _pps_sc(src_hbm, tgt_hbm, idx_hbm, ss_hbm, su_hbm, ts_hbm, tv_hbm,
            out_hbm, t_ss, t_su, t_ts, t_tv, idxbuf, rbuf, outbuf,
            sem0, sem1):
    c = lax.axis_index("c")
    s = lax.axis_index("s")
    poff = pl.multiple_of(s * PPT, PPT)
    lane = lax.iota(jnp.int32, L)
    sems = (sem0, sem1)

    boff = pl.multiple_of(c * BPC, BPC)
    pltpu.sync_copy(ss_hbm.at[pl.ds(boff, BPC)], t_ss)
    pltpu.sync_copy(su_hbm.at[pl.ds(boff, BPC)], t_su)
    pltpu.sync_copy(ts_hbm.at[pl.ds(boff, BPC)], t_ts)
    pltpu.sync_copy(tv_hbm.at[pl.ds(boff, BPC)], t_tv)

    def batch_body(bl, accs):
        acc1, acc2 = accs
        b = c * BPC + bl
        blv = jnp.full((L,), bl, jnp.int32)

        def issue(ch):
            slot = ch & 1
            cb = ch * CH
            hs = [pltpu.async_copy(
                src_hbm.at[b].at[idxbuf.at[0, pl.ds(cb, CH)]],
                rbuf.at[slot, 0], sems[slot])]
            for j in range(1, K2):
                hs.append(pltpu.async_copy(
                    tgt_hbm.at[b].at[idxbuf.at[j, pl.ds(cb, CH)]],
                    rbuf.at[slot, j], sems[slot]))
            return hs

        pltpu.sync_copy(idx_hbm.at[b, :, pl.ds(poff, PPT)], idxbuf)
        hs = issue(0)

        def compute(ch):
            slot = ch & 1
            cb = ch * CH
            i0 = idxbuf[0, pl.ds(cb, CH)]
            i1 = idxbuf[1, pl.ds(cb, CH)]
            sa = plsc.load_gather(t_ss, [blv, i0])
            ua = plsc.load_gather(t_su, [blv, i0])
            sp_ = plsc.load_gather(t_ts, [blv, i1])
            vp = plsc.load_gather(t_tv, [blv, i1])
            sns, vns = [], []
            for k in range(K):
                ik = idxbuf[2 + k, pl.ds(cb, CH)]
                sns.append(plsc.load_gather(t_ts, [blv, ik]))
                vns.append(plsc.load_gather(t_tv, [blv, ik]))
            cs = jnp.full((L,), slot, jnp.int32)
            cj = [jnp.full((L,), j, jnp.int32) for j in range(K2)]

            def dstep(d, acc):
                dp, dn = acc
                dv = (d + 8 * lane) & (D - 1)
                av = plsc.load_gather(rbuf, [cs, cj[0], lane, dv])
                dp = dp + av * plsc.load_gather(rbuf, [cs, cj[1], lane, dv])
                dn_out = []
                for k in range(K):
                    xk = plsc.load_gather(rbuf, [cs, cj[2 + k], lane, dv])
                    dn_out.append(dn[k] + av * xk)
                return dp, tuple(dn_out)

            zero = jnp.zeros((L,), jnp.float32)
            dp, dn = lax.fori_loop(0, D, dstep, (zero, (zero,) * K), unroll=2)
            base = ua + DEPS2
            sa2 = sa + sa
            d2p = base + vp - (sa2 * sp_) * dp
            dmin = base + vns[0] - (sa2 * sns[0]) * dn[0]
            for k in range(1, K):
                dmin = jnp.minimum(dmin, base + vns[k] - (sa2 * sns[k]) * dn[k])
            q = jnp.maximum(dmin, 1e-30)
            dd = q * _rsqrt16(q)
            hin = jnp.maximum(MARGIN - dd, 0.0)
            return d2p, hin * hin

        for ch in range(NCH):
            nxt = issue(ch + 1) if ch + 1 < NCH else None
            for h in hs:
                h.wait()
            d1, d2 = compute(ch)
            acc1 = acc1 + d1
            acc2 = acc2 + d2
            hs = nxt

        return acc1, acc2

    zero = jnp.zeros((L,), jnp.float32)
    acc1, acc2 = lax.fori_loop(0, BPC, batch_body, (zero, zero))
    outbuf[0, :] = acc1
    outbuf[1, :] = acc2
    pltpu.sync_copy(outbuf, out_hbm.at[c, s])


def kernel(src_feat, tgt_feat, neg_idxs):
    idx_t = jnp.transpose(neg_idxs.astype(jnp.int32), (0, 2, 1))
    ss, su, ts, tv = (a.reshape(B, M)
                      for a in _row_consts(src_feat, tgt_feat))
    parts = _pps_sc(src_feat, tgt_feat, idx_t, ss, su, ts, tv)
    return parts.sum() / jnp.float32(B * P)


# speedup vs baseline: 1.2449x; 1.2449x over previous
import functools

import jax
import jax.numpy as jnp
from jax import lax
from jax.experimental import pallas as pl
from jax.experimental.pallas import tpu as pltpu
from jax.experimental.pallas import tpu_sc as plsc

B, M, D = 8, 4096, 128
P, K2 = 2048, 10
K = K2 - 2
NC, NS, L = 2, 16, 16
BPC = B // NC
PPT = P // NS
CH = L
NCH = PPT // CH
EPS = 1e-6
DEPS2 = D * EPS * EPS
MARGIN = 0.5


def _consts_tc(src_ref, tgt_ref, ss_ref, su_ref, ts_ref, tv_ref):
    for tab, (ref, sref, uref) in enumerate(
            ((src_ref, ss_ref, su_ref), (tgt_ref, ts_ref, tv_ref))):
        x = ref[0]
        ones = jnp.ones((1, D), jnp.float32)
        q = jax.lax.dot_general(
            ones, x * x, (((1,), (1,)), ((), ())),
            preferred_element_type=jnp.float32)
        sv = jax.lax.dot_general(
            ones, x, (((1,), (1,)), ((), ())),
            preferred_element_type=jnp.float32)
        scale = jnp.minimum(lax.rsqrt(jnp.maximum(q, 1e-30)), 1e12)
        es = (2.0 * EPS) * scale * sv
        sq = scale * scale * q
        sref[0] = scale
        uref[0] = sq + es if tab == 0 else sq - es


MBLK = 1024

_row_consts = pl.pallas_call(
    _consts_tc,
    grid=(B, M // MBLK),
    in_specs=[pl.BlockSpec((1, MBLK, D), lambda b, m: (b, m, 0))] * 2,
    out_specs=[pl.BlockSpec((1, 1, MBLK), lambda b, m: (b, 0, m))] * 4,
    out_shape=[jax.ShapeDtypeStruct((B, 1, M), jnp.float32)] * 4,
)


def _rsqrt16(q):
    i = lax.bitcast_convert_type(q, jnp.int32)
    i = jnp.int32(0x5F3759DF) - (i >> 1)
    y = lax.bitcast_convert_type(i, jnp.float32)
    for _ in range(3):
        y = y * (1.5 - 0.5 * q * y * y)
    return y


_mesh = plsc.VectorSubcoreMesh(core_axis_name="c", subcore_axis_name="s")


@functools.partial(
    pl.kernel,
    out_type=jax.ShapeDtypeStruct((NC, NS, 2, L), jnp.float32),
    mesh=_mesh,
    compiler_params=pltpu.CompilerParams(needs_layout_passes=False),
    scratch_types=[
        pltpu.VMEM((BPC, M), jnp.float32),
        pltpu.VMEM((BPC, M), jnp.float32),
        pltpu.VMEM((BPC, M), jnp.float32),
        pltpu.VMEM((BPC, M), jnp.float32),
        pltpu.VMEM((K2, PPT), jnp.int32),
        pltpu.VMEM((2, K2, CH, D), jnp.float32),
        pltpu.VMEM((2, L), jnp.float32),
        pltpu.SemaphoreType.DMA,
        pltpu.SemaphoreType.DMA,
    ],
)
def _pps_sc(src_hbm, tgt_hbm, idx_hbm, ss_hbm, su_hbm, ts_hbm, tv_hbm,
            out_hbm, t_ss, t_su, t_ts, t_tv, idxbuf, rbuf, outbuf,
            sem0, sem1):
    c = lax.axis_index("c")
    s = lax.axis_index("s")
    poff = pl.multiple_of(s * PPT, PPT)
    lane = lax.iota(jnp.int32, L)
    sems = (sem0, sem1)

    boff = pl.multiple_of(c * BPC, BPC)
    pltpu.sync_copy(ss_hbm.at[pl.ds(boff, BPC)], t_ss)
    pltpu.sync_copy(su_hbm.at[pl.ds(boff, BPC)], t_su)
    pltpu.sync_copy(ts_hbm.at[pl.ds(boff, BPC)], t_ts)
    pltpu.sync_copy(tv_hbm.at[pl.ds(boff, BPC)], t_tv)

    def batch_body(bl, accs):
        acc1, acc2 = accs
        b = c * BPC + bl
        blv = jnp.full((L,), bl, jnp.int32)

        def issue(ch):
            slot = ch & 1
            cb = ch * CH
            hs = [pltpu.async_copy(
                src_hbm.at[b].at[idxbuf.at[0, pl.ds(cb, CH)]],
                rbuf.at[slot, 0], sems[slot])]
            for j in range(1, K2):
                hs.append(pltpu.async_copy(
                    tgt_hbm.at[b].at[idxbuf.at[j, pl.ds(cb, CH)]],
                    rbuf.at[slot, j], sems[slot]))
            return hs

        pltpu.sync_copy(idx_hbm.at[b, :, pl.ds(poff, PPT)], idxbuf)
        hs = issue(0)

        def compute(ch):
            slot = ch & 1
            cb = ch * CH
            i0 = idxbuf[0, pl.ds(cb, CH)]
            i1 = idxbuf[1, pl.ds(cb, CH)]
            sa = plsc.load_gather(t_ss, [blv, i0])
            ua = plsc.load_gather(t_su, [blv, i0])
            sp_ = plsc.load_gather(t_ts, [blv, i1])
            vp = plsc.load_gather(t_tv, [blv, i1])
            sns, vns = [], []
            for k in range(K):
                ik = idxbuf[2 + k, pl.ds(cb, CH)]
                sns.append(plsc.load_gather(t_ts, [blv, ik]))
                vns.append(plsc.load_gather(t_tv, [blv, ik]))
            cs = jnp.full((L,), slot, jnp.int32)
            cj = [jnp.full((L,), j, jnp.int32) for j in range(K2)]

            def dstep(d, acc):
                dp, dn = acc
                dv = (d + 8 * lane) & (D - 1)
                av = plsc.load_gather(rbuf, [cs, cj[0], lane, dv])
                dp = dp + av * plsc.load_gather(rbuf, [cs, cj[1], lane, dv])
                dn_out = []
                for k in range(K):
                    xk = plsc.load_gather(rbuf, [cs, cj[2 + k], lane, dv])
                    dn_out.append(dn[k] + av * xk)
                return dp, tuple(dn_out)

            zero = jnp.zeros((L,), jnp.float32)
            dp, dn = lax.fori_loop(0, D, dstep, (zero, (zero,) * K), unroll=2)
            base = ua + DEPS2
            sa2 = sa + sa
            d2p = base + vp - (sa2 * sp_) * dp
            dmin = base + vns[0] - (sa2 * sns[0]) * dn[0]
            for k in range(1, K):
                dmin = jnp.minimum(dmin, base + vns[k] - (sa2 * sns[k]) * dn[k])
            q = jnp.maximum(dmin, 1e-30)
            dd = q * _rsqrt16(q)
            hin = jnp.maximum(MARGIN - dd, 0.0)
            return d2p, hin * hin

        for ch in range(NCH):
            nxt = issue(ch + 1) if ch + 1 < NCH else None
            for h in hs:
                h.wait()
            d1, d2 = compute(ch)
            acc1 = acc1 + d1
            acc2 = acc2 + d2
            hs = nxt

        return acc1, acc2

    zero = jnp.zeros((L,), jnp.float32)
    acc1, acc2 = lax.fori_loop(0, BPC, batch_body, (zero, zero))
    outbuf[0, :] = acc1
    outbuf[1, :] = acc2
    pltpu.sync_copy(outbuf, out_hbm.at[c, s])


def kernel(src_feat, tgt_feat, neg_idxs):
    idx_t = jnp.transpose(neg_idxs.astype(jnp.int32), (0, 2, 1))
    ss, su, ts, tv = (a.reshape(B, M)
                      for a in _row_consts(src_feat, tgt_feat))
    parts = _pps_sc(src_feat, tgt_feat, idx_t, ss, su, ts, tv)
    return parts.sum() / jnp.float32(B * P)
